# baseline (speedup 1.0000x reference)
import functools
import math

import jax
import jax.numpy as jnp
from jax import lax
from jax.experimental import pallas as pl
from jax.experimental.pallas import tpu as pltpu
from jax.experimental.pallas import tpu_sc as plsc

B, S, H, G, K = 16, 1, 16, 1, 2048
DIM, TAIL = 512, 64
D = DIM + TAIL
SKV = 8192
SM_SCALE = 1.0 / math.sqrt(D)

NC, NS = 2, 16
LANES = 16


def _sc_histogram(idx2d):
    mesh = plsc.VectorSubcoreMesh(core_axis_name="c", subcore_axis_name="s")

    @functools.partial(
        pl.kernel,
        out_type=jax.ShapeDtypeStruct((B, 1, SKV), jnp.float32),
        mesh=mesh,
        scratch_types=[
            pltpu.VMEM((K,), jnp.int32),
            pltpu.VMEM((SKV,), jnp.float32),
        ],
        compiler_params=pltpu.CompilerParams(
            use_tc_tiling_on_sc=False, needs_layout_passes=False),
    )
    def hist_kernel(idx_hbm, out_hbm, idx_v, cnt_v):
        wid = lax.axis_index("s") * NC + lax.axis_index("c")

        @pl.when(wid < B)
        def _():
            pltpu.sync_copy(idx_hbm.at[wid], idx_v)
            zeros = jnp.zeros((LANES,), jnp.float32)
            for j in range(SKV // LANES):
                cnt_v[pl.ds(j * LANES, LANES)] = zeros
            ones = jnp.ones((LANES,), jnp.float32)
            for j in range(K // LANES):
                ids = idx_v[pl.ds(j * LANES, LANES)]
                plsc.addupdate_scatter(cnt_v, [ids], ones)
            pltpu.sync_copy(cnt_v, out_hbm.at[wid, 0])

    return hist_kernel(idx2d)


W = 2048
NCH = SKV // W


def _attn_body(q_ref, kvt_ref, cnt_ref, o_ref, m_ref, l_ref, acc_ref):
    c = pl.program_id(1)
    qb = q_ref[0]
    kt = kvt_ref[0]
    cnt = cnt_ref[0]
    s = lax.dot_general(qb, kt, (((1,), (0,)), ((), ())),
                        preferred_element_type=jnp.float32) * SM_SCALE
    m_c = jnp.max(s, axis=1, keepdims=True)

    @pl.when(c == 0)
    def _init():
        m_ref[...] = m_c
        e = jnp.exp(s - m_c) * cnt
        l_ref[...] = jnp.sum(e, axis=1, keepdims=True)
        acc_ref[...] = lax.dot_general(
            e, kt[:DIM, :], (((1,), (1,)), ((), ())),
            preferred_element_type=jnp.float32)

    @pl.when(c > 0)
    def _step():
        m_prev = m_ref[...]
        m_new = jnp.maximum(m_prev, m_c)
        alpha = jnp.exp(m_prev - m_new)
        e = jnp.exp(s - m_new) * cnt
        m_ref[...] = m_new
        l_ref[...] = l_ref[...] * alpha + jnp.sum(e, axis=1, keepdims=True)
        acc_ref[...] = acc_ref[...] * alpha + lax.dot_general(
            e, kt[:DIM, :], (((1,), (1,)), ((), ())),
            preferred_element_type=jnp.float32)

    @pl.when(c == NCH - 1)
    def _fini():
        o_ref[0] = acc_ref[...] / l_ref[...]


def _tc_attention(q3, kvt3, counts):
    return pl.pallas_call(
        _attn_body,
        grid=(B, NCH),
        in_specs=[
            pl.BlockSpec((1, H, D), lambda b, c: (b, 0, 0)),
            pl.BlockSpec((1, D, W), lambda b, c: (b, 0, c)),
            pl.BlockSpec((1, 1, W), lambda b, c: (b, 0, c)),
        ],
        out_specs=pl.BlockSpec((1, H, DIM), lambda b, c: (b, 0, 0)),
        out_shape=jax.ShapeDtypeStruct((B, H, DIM), jnp.float32),
        scratch_shapes=[
            pltpu.VMEM((H, 1), jnp.float32),
            pltpu.VMEM((H, 1), jnp.float32),
            pltpu.VMEM((H, DIM), jnp.float32),
        ],
    )(q3, kvt3, counts)


def kernel(q, kv, indices):
    counts = _sc_histogram(indices.reshape(B, K))
    kvt3 = jnp.transpose(kv, (0, 2, 3, 1)).reshape(B, D, SKV)
    out = _tc_attention(q.reshape(B, H, D), kvt3, counts)
    return out.reshape(B, S, H, DIM)


# speedup vs baseline: 1.2166x; 1.2166x over previous
import functools
import math

import jax
import jax.numpy as jnp
from jax import lax
from jax.experimental import pallas as pl
from jax.experimental.pallas import tpu as pltpu
from jax.experimental.pallas import tpu_sc as plsc

B, S, H, G, K = 16, 1, 16, 1, 2048
DIM, TAIL = 512, 64
D = DIM + TAIL
SKV = 8192
SM_SCALE = 1.0 / math.sqrt(D)

NC, NS = 2, 16
LANES = 16


def _sc_histogram(idx2d):
    mesh = plsc.VectorSubcoreMesh(core_axis_name="c", subcore_axis_name="s")

    @functools.partial(
        pl.kernel,
        out_type=jax.ShapeDtypeStruct((B, 1, SKV), jnp.float32),
        mesh=mesh,
        scratch_types=[
            pltpu.VMEM((K,), jnp.int32),
            pltpu.VMEM((SKV,), jnp.float32),
        ],
        compiler_params=pltpu.CompilerParams(
            use_tc_tiling_on_sc=False, needs_layout_passes=False),
    )
    def hist_kernel(idx_hbm, out_hbm, idx_v, cnt_v):
        wid = lax.axis_index("s") * NC + lax.axis_index("c")

        @pl.when(wid < B)
        def _():
            pltpu.sync_copy(idx_hbm.at[wid], idx_v)
            zeros = jnp.zeros((LANES,), jnp.float32)
            for j in range(SKV // LANES):
                cnt_v[pl.ds(j * LANES, LANES)] = zeros
            ones = jnp.ones((LANES,), jnp.float32)
            for j in range(K // LANES):
                ids = idx_v[pl.ds(j * LANES, LANES)]
                plsc.addupdate_scatter(cnt_v, [ids], ones)
            pltpu.sync_copy(cnt_v, out_hbm.at[wid, 0])

    return hist_kernel(idx2d)


W2 = SKV // 2


def _attn_body(q_ref, ka_ref, kb_ref, cnt_ref, o_ref):
    qb = q_ref[0]
    ka = ka_ref[0]
    kb = kb_ref[0]
    cnt = cnt_ref[0]
    sa = lax.dot_general(qb, ka, (((1,), (0,)), ((), ())),
                         preferred_element_type=jnp.float32) * SM_SCALE
    sb = lax.dot_general(qb, kb, (((1,), (0,)), ((), ())),
                         preferred_element_type=jnp.float32) * SM_SCALE
    m = jnp.maximum(jnp.max(sa, axis=1, keepdims=True),
                    jnp.max(sb, axis=1, keepdims=True))
    ea = jnp.exp(sa - m) * cnt[:, :W2]
    eb = jnp.exp(sb - m) * cnt[:, W2:]
    den = (jnp.sum(ea, axis=1, keepdims=True)
           + jnp.sum(eb, axis=1, keepdims=True))
    o = (lax.dot_general(ea, ka[:DIM, :], (((1,), (1,)), ((), ())),
                         preferred_element_type=jnp.float32)
         + lax.dot_general(eb, kb[:DIM, :], (((1,), (1,)), ((), ())),
                           preferred_element_type=jnp.float32))
    o_ref[0] = o / den


def _tc_attention(q3, kvt3, counts):
    return pl.pallas_call(
        _attn_body,
        grid=(B,),
        in_specs=[
            pl.BlockSpec((1, H, D), lambda b: (b, 0, 0)),
            pl.BlockSpec((1, D, W2), lambda b: (b, 0, 0)),
            pl.BlockSpec((1, D, W2), lambda b: (b, 0, 1)),
            pl.BlockSpec((1, 1, SKV), lambda b: (b, 0, 0)),
        ],
        out_specs=pl.BlockSpec((1, H, DIM), lambda b: (b, 0, 0)),
        out_shape=jax.ShapeDtypeStruct((B, H, DIM), jnp.float32),
    )(q3, kvt3, kvt3, counts)


def kernel(q, kv, indices):
    counts = _sc_histogram(indices.reshape(B, K))
    kvt3 = jnp.transpose(kv, (0, 2, 3, 1)).reshape(B, D, SKV)
    out = _tc_attention(q.reshape(B, H, D), kvt3, counts)
    return out.reshape(B, S, H, DIM)
